# Initial kernel scaffold; baseline (speedup 1.0000x reference)
import jax
import jax.numpy as jnp
from jax.experimental import pallas as pl


def kernel(x, eigvals, eigvecs, edge_attr, edge_index, batch, token_index, real_edges, degrees, atom_W, atom_b, bond_W, bond_b, phi_W1, phi_b1, phi_W2, phi_b2, rho_W1, rho_b1, rho_W2, rho_b2, eps, eig_W, eig_b, deg_table, deg_W, deg_b, node_W, node_b, graph_token):
    raise NotImplementedError("write your pallas kernel here")


# speedup vs baseline: 10.5811x; 10.5811x over previous
import functools

import jax
import jax.numpy as jnp
from jax import lax
from jax.experimental import pallas as pl
from jax.experimental.pallas import tpu as pltpu
from jax.experimental.pallas import tpu_sc as plsc

HIGH = lax.Precision.HIGHEST


def _tables_body(x_ref, ev_ref, el_ref, deg_ref,
                 atom_W_ref, atom_b_ref, phi_W1_ref, phi_b1_ref, phi_W2_ref,
                 phi_b2_ref, rho_W1_ref, rho_b1_ref, rho_W2_ref, rho_b2_ref,
                 eps_ref, eig_W_ref, deg_table_ref, deg_W_ref, node_W_ref,
                 a0_ref, a1_ref):
    f32 = jnp.float32
    relu = lambda v: jnp.maximum(v, 0.0)
    K = ev_ref.shape[1]
    ED = phi_W2_ref.shape[1]
    D = x_ref.shape[1]
    DD = deg_table_ref.shape[1]

    x_enc = jnp.dot(x_ref[...], atom_W_ref[...], precision=HIGH,
                    preferred_element_type=f32) + atom_b_ref[...]

    ev = ev_ref[...]
    el = el_ref[...] + eps_ref[...]
    w1a = phi_W1_ref[0:1, :]
    w1b = phi_W1_ref[1:2, :]
    b1 = phi_b1_ref[...]
    acc = jnp.zeros((x_ref.shape[0], ED), dtype=f32)
    for k in range(K):
        h = relu(ev[:, k:k + 1] * w1a + el[:, k:k + 1] * w1b + b1)
        acc = acc + relu(jnp.dot(h, phi_W2_ref[...], precision=HIGH,
                                 preferred_element_type=f32) + phi_b2_ref[...])
    l = relu(jnp.dot(acc, rho_W1_ref[...], precision=HIGH,
                     preferred_element_type=f32) + rho_b1_ref[...])
    l = relu(jnp.dot(l, rho_W2_ref[...], precision=HIGH,
                     preferred_element_type=f32) + rho_b2_ref[...])

    iota = lax.broadcasted_iota(jnp.int32, (1, deg_table_ref.shape[0]), 1)
    oh = (deg_ref[...] == iota).astype(f32)
    demb = jnp.dot(oh, deg_table_ref[...], precision=HIGH,
                   preferred_element_type=f32)

    a0_ref[...] = (
        jnp.dot(x_enc, node_W_ref[:D, :], precision=HIGH, preferred_element_type=f32)
        + jnp.dot(l, eig_W_ref[:ED, :], precision=HIGH, preferred_element_type=f32)
        + jnp.dot(demb, deg_W_ref[:DD, :], precision=HIGH, preferred_element_type=f32))
    a1_ref[...] = (
        jnp.dot(x_enc, node_W_ref[D:, :], precision=HIGH, preferred_element_type=f32)
        + jnp.dot(l, eig_W_ref[ED:, :], precision=HIGH, preferred_element_type=f32)
        + jnp.dot(demb, deg_W_ref[DD:, :], precision=HIGH, preferred_element_type=f32))


def _bond_body(ea_ref, bond_W_ref, btot_ref, out_ref):
    out_ref[...] = jnp.dot(ea_ref[...], bond_W_ref[...], precision=HIGH,
                           preferred_element_type=jnp.float32) + btot_ref[...]


def _make_sc_gather(T, D, NC, NS):
    NW = NC * NS
    MACRO = 640
    SUB = 80
    NSUB = MACRO // SUB
    NCHUNK = T // MACRO
    assert NCHUNK * MACRO == T and NSUB * SUB == MACRO

    mesh = plsc.VectorSubcoreMesh(core_axis_name="c", subcore_axis_name="s")

    @functools.partial(
        pl.kernel, mesh=mesh,
        out_type=jax.ShapeDtypeStruct((T + 1, 1, D), jnp.float32),
        scratch_types=[
            pltpu.VMEM((NSUB, SUB), jnp.int32),
            pltpu.VMEM((NSUB, SUB), jnp.int32),
            pltpu.VMEM((MACRO, 1, D), jnp.float32),
            pltpu.SemaphoreType.DMA,
        ],
    )
    def sc_gather(a0_hbm, a1_hbm, bond_hbm, idx0_hbm, idx1_hbm, gt_hbm,
                  out_hbm, idx0_v, idx1_v, tok_v, sem):
        wid = lax.axis_index("s") * NC + lax.axis_index("c")

        @pl.when(wid == 0)
        def _():
            pltpu.sync_copy(gt_hbm, out_hbm.at[pl.ds(0, 1)])

        ntrips = jnp.where(wid < (NCHUNK % NW), NCHUNK // NW + 1, NCHUNK // NW)

        def macro(j, carry):
            c = j * NW + wid
            base = c * MACRO
            irow = c * NSUB
            pltpu.sync_copy(bond_hbm.at[pl.ds(base, MACRO)], tok_v)
            pltpu.sync_copy(idx0_hbm.at[pl.ds(irow, NSUB)], idx0_v)
            pltpu.sync_copy(idx1_hbm.at[pl.ds(irow, NSUB)], idx1_v)

            def sub(k, cc):
                dst = tok_v.at[pl.ds(k * SUB, SUB)]
                pltpu.async_copy(a0_hbm.at[idx0_v.at[k]], dst, sem, add=True)
                pltpu.async_copy(a1_hbm.at[idx1_v.at[k]], dst, sem, add=True)
                return cc

            lax.fori_loop(0, NSUB, sub, 0)
            pltpu.make_async_copy(bond_hbm.at[pl.ds(0, MACRO)], tok_v, sem).wait()
            pltpu.make_async_copy(bond_hbm.at[pl.ds(0, MACRO)], tok_v, sem).wait()
            pltpu.sync_copy(tok_v, out_hbm.at[pl.ds(base + 1, MACRO)])
            return carry

        lax.fori_loop(0, ntrips, macro, 0)

    return sc_gather


def kernel(x, eigvals, eigvecs, edge_attr, edge_index, batch, token_index,
           real_edges, degrees, atom_W, atom_b, bond_W, bond_b, phi_W1, phi_b1,
           phi_W2, phi_b2, rho_W1, rho_b1, rho_W2, rho_b2, eps, eig_W, eig_b,
           deg_table, deg_W, deg_b, node_W, node_b, graph_token):
    N, D = x.shape
    K = eigvals.shape[1]
    T = token_index.shape[1]
    E, EA = edge_attr.shape
    ED = rho_W2.shape[1]
    DD = deg_table.shape[1]
    MAXDEG = deg_table.shape[0]
    f32 = jnp.float32

    NB = 2000
    rep = lambda shape: pl.BlockSpec(shape, lambda i: (0, 0))
    tables = pl.pallas_call(
        _tables_body,
        grid=(N // NB,),
        in_specs=[
            pl.BlockSpec((NB, D), lambda i: (i, 0)),
            pl.BlockSpec((NB, K), lambda i: (i, 0)),
            pl.BlockSpec((NB, K), lambda i: (i, 0)),
            pl.BlockSpec((NB, 1), lambda i: (i, 0)),
            rep((D, D)), rep((1, D)),
            rep((2, 2 * ED)), rep((1, 2 * ED)),
            rep((2 * ED, ED)), rep((1, ED)),
            rep((ED, 2 * ED)), rep((1, 2 * ED)),
            rep((2 * ED, ED)), rep((1, ED)),
            rep((1, K)),
            rep((2 * ED, D)),
            rep((MAXDEG, DD)), rep((2 * DD, D)),
            rep((2 * D, D)),
        ],
        out_specs=[
            pl.BlockSpec((NB, D), lambda i: (i, 0)),
            pl.BlockSpec((NB, D), lambda i: (i, 0)),
        ],
        out_shape=[
            jax.ShapeDtypeStruct((N, D), f32),
            jax.ShapeDtypeStruct((N, D), f32),
        ],
    )
    a0, a1 = tables(
        x, eigvecs, eigvals, degrees.reshape(N, 1),
        atom_W, atom_b.reshape(1, D),
        phi_W1, phi_b1.reshape(1, 2 * ED), phi_W2, phi_b2.reshape(1, ED),
        rho_W1, rho_b1.reshape(1, 2 * ED), rho_W2, rho_b2.reshape(1, ED),
        eps, eig_W, deg_table, deg_W, node_W)

    btot = (bond_b + eig_b + node_b + deg_b).reshape(1, D)
    TB = 4000
    bond = pl.pallas_call(
        _bond_body,
        grid=(T // TB,),
        in_specs=[
            pl.BlockSpec((TB, EA), lambda i: (i, 0)),
            rep((EA, D)), rep((1, D)),
        ],
        out_specs=pl.BlockSpec((TB, D), lambda i: (i, 0)),
        out_shape=jax.ShapeDtypeStruct((T, D), f32),
    )(edge_attr, bond_W, btot)

    info = plsc.get_sparse_core_info()
    NC, NS = info.num_cores, info.num_subcores
    SUB = 80
    idx0 = token_index[0].reshape(T // SUB, SUB)
    idx1 = token_index[1].reshape(T // SUB, SUB)
    sc_gather = _make_sc_gather(T, D, NC, NS)
    out3d = sc_gather(a0.reshape(N, 1, D), a1.reshape(N, 1, D),
                      bond.reshape(T, 1, D), idx0, idx1,
                      graph_token.reshape(1, 1, D))

    out = out3d.reshape(1, T + 1, D)
    padding_mask = jnp.ones((1, T + 1), dtype=bool)
    return (out, padding_mask)
